# baseline (speedup 1.0000x reference)
import functools

import jax
import jax.numpy as jnp
from jax import lax
from jax.experimental import pallas as pl
from jax.experimental.pallas import tpu as pltpu
from jax.experimental.pallas import tpu_sc as plsc

B, NFIELD, NFEAT, SQL_NEMB, DATA_NEMB = 1024, 26, 100000, 16, 64
K, C, H, OUT = 8, 2, 1024, 1
CARD = NFIELD + NFEAT + 1
IN_SZ = NFIELD * DATA_NEMB
G_IN = NFIELD * SQL_NEMB

_NC, _NS = 2, 16
_NW = _NC * _NS
_NIDX = B * NFIELD
_BPW = _NIDX // _NW

_MM_PREC = lax.Precision.DEFAULT


@functools.lru_cache(maxsize=None)
def _build_sc_gather(nemb):
    @functools.partial(
        pl.kernel,
        out_type=jax.ShapeDtypeStruct((_NIDX, nemb), jnp.float32),
        mesh=plsc.VectorSubcoreMesh(
            core_axis_name="c", subcore_axis_name="s",
            num_cores=_NC, num_subcores=_NS,
        ),
        scratch_types=[
            pltpu.VMEM((_BPW,), jnp.int32),
            pltpu.VMEM((_BPW, nemb), jnp.float32),
            pltpu.SemaphoreType.DMA,
        ],
        compiler_params=pltpu.CompilerParams(use_tc_tiling_on_sc=False),
    )
    def _sc_gather(idx_hbm, tab_hbm, out_hbm, idx_v, rows_v, sem):
        wid = lax.axis_index("s") * _NC + lax.axis_index("c")
        base = wid * _BPW
        pltpu.sync_copy(idx_hbm.at[pl.ds(base, _BPW)], idx_v)
        pltpu.async_copy(tab_hbm.at[idx_v], rows_v, sem).wait()
        pltpu.sync_copy(rows_v, out_hbm.at[pl.ds(base, _BPW)])

    return _sc_gather


def _gate_body(s26_ref, wg1_ref, bg1_ref, wg2_ref, bg2_ref,
               gates_ref, loss_ref):
    gh = jnp.dot(s26_ref[0], wg1_ref[0:SQL_NEMB, :], precision=_MM_PREC)
    for f in range(1, NFIELD):
        gh = gh + jnp.dot(s26_ref[f],
                          wg1_ref[f * SQL_NEMB:(f + 1) * SQL_NEMB, :],
                          precision=_MM_PREC)
    gh = jnp.maximum(gh + bg1_ref[...], 0.0)
    logits = jnp.dot(gh, wg2_ref[...], precision=_MM_PREC) + bg2_ref[...]
    mx = jnp.max(logits, axis=1, keepdims=True)
    e = jnp.exp(logits - mx)
    gate = e / jnp.sum(e, axis=1, keepdims=True)

    idx = lax.broadcasted_iota(jnp.int32, (B, K), 1)
    m1 = jnp.max(gate, axis=1, keepdims=True)
    i1 = jnp.min(jnp.where(gate == m1, idx, K), axis=1, keepdims=True)
    rest = jnp.where(idx == i1, -jnp.inf, gate)
    m2 = jnp.max(rest, axis=1, keepdims=True)
    i2 = jnp.min(jnp.where(rest == m2, idx, K), axis=1, keepdims=True)
    keep = (idx == i1) | (idx == i2)
    gates = jnp.where(keep, gate, 0.0) / (m1 + m2 + 1e-9)
    gates_ref[...] = gates

    imp = jnp.sum(gates, axis=0, keepdims=True)
    mi = jnp.mean(imp)
    vi = jnp.mean((imp - mi) ** 2)
    loss_ref[...] = jnp.reshape(vi / (mi * mi + 1e-10), (1, 1))


_gate_call = pl.pallas_call(
    _gate_body,
    out_shape=(
        jax.ShapeDtypeStruct((B, K), jnp.float32),
        jax.ShapeDtypeStruct((1, 1), jnp.float32),
    ),
)


def _dot_bf16(a, bmat):
    return lax.dot_general(
        a.astype(jnp.bfloat16), bmat.astype(jnp.bfloat16),
        (((1,), (0,)), ((), ())), preferred_element_type=jnp.float32)


def _bn_relu(z, g, b):
    m = jnp.mean(z, axis=0, keepdims=True)
    v = jnp.mean((z - m) ** 2, axis=0, keepdims=True)
    return jnp.maximum((z - m) * lax.rsqrt(v + 1e-5) * g + b, 0.0)


def _expert_body(gates_ref, x_emb_ref, wf1_ref, bf1_ref, g1_ref, be1_ref,
                 wf2_ref, bf2_ref, g2_ref, be2_ref,
                 wp1_ref, bp1_ref, gp1_ref, bep1_ref, wp2_ref, bp2_ref,
                 out_ref, y_ref):
    k = pl.program_id(0)

    @pl.when(k < K)
    def _():
        z = _dot_bf16(x_emb_ref[...], wf1_ref[0])
        h = _bn_relu(z + bf1_ref[0], g1_ref[0], be1_ref[0])
        z2 = _dot_bf16(h, wf2_ref[0])
        o = _bn_relu(z2 + bf2_ref[0], g2_ref[0], be2_ref[0])
        onehot = (lax.broadcasted_iota(jnp.int32, (K, 1), 0) == k
                  ).astype(jnp.float32)
        gcol = jnp.dot(gates_ref[...], onehot,
                       precision=lax.Precision.HIGHEST)
        contrib = o * gcol

        @pl.when(k == 0)
        def _():
            y_ref[...] = contrib

        @pl.when(k > 0)
        def _():
            y_ref[...] = y_ref[...] + contrib

    @pl.when(k == K)
    def _():
        z3 = _dot_bf16(y_ref[...], wp1_ref[...])
        p = _bn_relu(z3 + bp1_ref[...], gp1_ref[...], bep1_ref[...])
        out_ref[...] = (jnp.dot(p, wp2_ref[...], precision=_MM_PREC)
                        + bp2_ref[...])


def _ei(k):
    return jnp.minimum(k, K - 1)


_expert_call = pl.pallas_call(
    _expert_body,
    grid=(K + 1,),
    in_specs=[
        pl.BlockSpec((B, K), lambda k: (0, 0)),
        pl.BlockSpec((B, IN_SZ), lambda k: (0, 0)),
        pl.BlockSpec((1, IN_SZ, H), lambda k: (_ei(k), 0, 0)),
        pl.BlockSpec((1, 1, H), lambda k: (_ei(k), 0, 0)),
        pl.BlockSpec((1, 1, H), lambda k: (_ei(k), 0, 0)),
        pl.BlockSpec((1, 1, H), lambda k: (_ei(k), 0, 0)),
        pl.BlockSpec((1, H, H), lambda k: (_ei(k), 0, 0)),
        pl.BlockSpec((1, 1, H), lambda k: (_ei(k), 0, 0)),
        pl.BlockSpec((1, 1, H), lambda k: (_ei(k), 0, 0)),
        pl.BlockSpec((1, 1, H), lambda k: (_ei(k), 0, 0)),
        pl.BlockSpec((H, H), lambda k: (0, 0)),
        pl.BlockSpec((1, H), lambda k: (0, 0)),
        pl.BlockSpec((1, H), lambda k: (0, 0)),
        pl.BlockSpec((1, H), lambda k: (0, 0)),
        pl.BlockSpec((H, OUT), lambda k: (0, 0)),
        pl.BlockSpec((1, OUT), lambda k: (0, 0)),
    ],
    out_specs=pl.BlockSpec((B, OUT), lambda k: (0, 0)),
    out_shape=jax.ShapeDtypeStruct((B, OUT), jnp.float32),
    scratch_shapes=[pltpu.VMEM((B, H), jnp.float32)],
    compiler_params=pltpu.CompilerParams(vmem_limit_bytes=128 * 1024 * 1024),
)


def kernel(x, sql, sql_table, input_table, Wg1, bg1, Wg2, bg2, Wf1, bf1,
           g1, be1, Wf2, bf2, g2, be2, Wp1, bp1, gp1, bep1, Wp2, bp2):
    xf = x.reshape(_NIDX).astype(jnp.int32)
    sf = sql.astype(jnp.int32).T.reshape(_NIDX)
    xrows = _build_sc_gather(DATA_NEMB)(xf, input_table)
    srows = _build_sc_gather(SQL_NEMB)(sf, sql_table)
    x_emb = xrows.reshape(B, IN_SZ)
    s26 = srows.reshape(NFIELD, B, SQL_NEMB)

    gates, loss = _gate_call(s26, Wg1, bg1.reshape(1, H),
                             Wg2, bg2.reshape(1, K))
    out2 = _expert_call(gates, x_emb, Wf1, bf1.reshape(K, 1, H),
                        g1.reshape(K, 1, H), be1.reshape(K, 1, H), Wf2,
                        bf2.reshape(K, 1, H), g2.reshape(K, 1, H),
                        be2.reshape(K, 1, H), Wp1, bp1.reshape(1, H),
                        gp1.reshape(1, H), bep1.reshape(1, H), Wp2,
                        bp2.reshape(1, OUT))
    return out2.reshape(B), loss.reshape(())


# speedup vs baseline: 1.0988x; 1.0988x over previous
import functools

import jax
import jax.numpy as jnp
from jax import lax
from jax.experimental import pallas as pl
from jax.experimental.pallas import tpu as pltpu
from jax.experimental.pallas import tpu_sc as plsc

B, NFIELD, NFEAT, SQL_NEMB, DATA_NEMB = 1024, 26, 100000, 16, 64
K, C, H, OUT = 8, 2, 1024, 1
CARD = NFIELD + NFEAT + 1
IN_SZ = NFIELD * DATA_NEMB
G_IN = NFIELD * SQL_NEMB

_NC, _NS = 2, 16
_NW = _NC * _NS
_NIDX = B * NFIELD
_BPW = _NIDX // _NW

_MM_PREC = lax.Precision.DEFAULT


@functools.lru_cache(maxsize=None)
def _build_sc_gather(nemb):
    @functools.partial(
        pl.kernel,
        out_type=jax.ShapeDtypeStruct((_NIDX, nemb), jnp.float32),
        mesh=plsc.VectorSubcoreMesh(
            core_axis_name="c", subcore_axis_name="s",
            num_cores=_NC, num_subcores=_NS,
        ),
        scratch_types=[
            pltpu.VMEM((_BPW,), jnp.int32),
            pltpu.VMEM((_BPW, nemb), jnp.float32),
            pltpu.SemaphoreType.DMA,
        ],
        compiler_params=pltpu.CompilerParams(use_tc_tiling_on_sc=False),
    )
    def _sc_gather(idx_hbm, tab_hbm, out_hbm, idx_v, rows_v, sem):
        wid = lax.axis_index("s") * _NC + lax.axis_index("c")
        base = wid * _BPW
        pltpu.sync_copy(idx_hbm.at[pl.ds(base, _BPW)], idx_v)
        pltpu.async_copy(tab_hbm.at[idx_v], rows_v, sem).wait()
        pltpu.sync_copy(rows_v, out_hbm.at[pl.ds(base, _BPW)])

    return _sc_gather


def _gate_body(sql_emb_ref, wg1_ref, bg1_ref, wg2_ref, bg2_ref,
               gates_ref, loss_ref):
    gh = jnp.dot(sql_emb_ref[...], wg1_ref[...], precision=_MM_PREC)
    gh = jnp.maximum(gh + bg1_ref[...], 0.0)
    logits = jnp.dot(gh, wg2_ref[...], precision=_MM_PREC) + bg2_ref[...]
    mx = jnp.max(logits, axis=1, keepdims=True)
    e = jnp.exp(logits - mx)
    gate = e / jnp.sum(e, axis=1, keepdims=True)

    idx = lax.broadcasted_iota(jnp.int32, (B, K), 1)
    m1 = jnp.max(gate, axis=1, keepdims=True)
    i1 = jnp.min(jnp.where(gate == m1, idx, K), axis=1, keepdims=True)
    rest = jnp.where(idx == i1, -jnp.inf, gate)
    m2 = jnp.max(rest, axis=1, keepdims=True)
    i2 = jnp.min(jnp.where(rest == m2, idx, K), axis=1, keepdims=True)
    keep = (idx == i1) | (idx == i2)
    gates = jnp.where(keep, gate, 0.0) / (m1 + m2 + 1e-9)
    gates_ref[...] = gates

    imp = jnp.sum(gates, axis=0, keepdims=True)
    mi = jnp.mean(imp)
    vi = jnp.mean((imp - mi) ** 2)
    loss_ref[...] = jnp.reshape(vi / (mi * mi + 1e-10), (1, 1))


_gate_call = pl.pallas_call(
    _gate_body,
    out_shape=(
        jax.ShapeDtypeStruct((B, K), jnp.float32),
        jax.ShapeDtypeStruct((1, 1), jnp.float32),
    ),
)


def _dot_bf16(a, bmat):
    return lax.dot_general(
        a.astype(jnp.bfloat16), bmat.astype(jnp.bfloat16),
        (((1,), (0,)), ((), ())), preferred_element_type=jnp.float32)


def _bn_relu(z, g, b):
    m = jnp.mean(z, axis=0, keepdims=True)
    v = jnp.mean((z - m) ** 2, axis=0, keepdims=True)
    return jnp.maximum((z - m) * lax.rsqrt(v + 1e-5) * g + b, 0.0)


def _expert_body(gates_ref, x_emb_ref, wf1_ref, bf1_ref, g1_ref, be1_ref,
                 wf2_ref, bf2_ref, g2_ref, be2_ref,
                 wp1_ref, bp1_ref, gp1_ref, bep1_ref, wp2_ref, bp2_ref,
                 out_ref, y_ref):
    k = pl.program_id(0)

    @pl.when(k < K)
    def _():
        z = _dot_bf16(x_emb_ref[...], wf1_ref[0])
        h = _bn_relu(z + bf1_ref[0], g1_ref[0], be1_ref[0])
        z2 = _dot_bf16(h, wf2_ref[0])
        o = _bn_relu(z2 + bf2_ref[0], g2_ref[0], be2_ref[0])
        onehot = (lax.broadcasted_iota(jnp.int32, (K, 1), 0) == k
                  ).astype(jnp.float32)
        gcol = jnp.dot(gates_ref[...], onehot,
                       precision=lax.Precision.HIGHEST)
        contrib = o * gcol

        @pl.when(k == 0)
        def _():
            y_ref[...] = contrib

        @pl.when(k > 0)
        def _():
            y_ref[...] = y_ref[...] + contrib

    @pl.when(k == K)
    def _():
        z3 = _dot_bf16(y_ref[...], wp1_ref[...])
        p = _bn_relu(z3 + bp1_ref[...], gp1_ref[...], bep1_ref[...])
        out_ref[...] = (jnp.dot(p, wp2_ref[...], precision=_MM_PREC)
                        + bp2_ref[...])


def _ei(k):
    return jnp.minimum(k, K - 1)


_expert_call = pl.pallas_call(
    _expert_body,
    grid=(K + 1,),
    in_specs=[
        pl.BlockSpec((B, K), lambda k: (0, 0)),
        pl.BlockSpec((B, IN_SZ), lambda k: (0, 0)),
        pl.BlockSpec((1, IN_SZ, H), lambda k: (_ei(k), 0, 0)),
        pl.BlockSpec((1, 1, H), lambda k: (_ei(k), 0, 0)),
        pl.BlockSpec((1, 1, H), lambda k: (_ei(k), 0, 0)),
        pl.BlockSpec((1, 1, H), lambda k: (_ei(k), 0, 0)),
        pl.BlockSpec((1, H, H), lambda k: (_ei(k), 0, 0)),
        pl.BlockSpec((1, 1, H), lambda k: (_ei(k), 0, 0)),
        pl.BlockSpec((1, 1, H), lambda k: (_ei(k), 0, 0)),
        pl.BlockSpec((1, 1, H), lambda k: (_ei(k), 0, 0)),
        pl.BlockSpec((H, H), lambda k: (0, 0)),
        pl.BlockSpec((1, H), lambda k: (0, 0)),
        pl.BlockSpec((1, H), lambda k: (0, 0)),
        pl.BlockSpec((1, H), lambda k: (0, 0)),
        pl.BlockSpec((H, OUT), lambda k: (0, 0)),
        pl.BlockSpec((1, OUT), lambda k: (0, 0)),
    ],
    out_specs=pl.BlockSpec((B, OUT), lambda k: (0, 0)),
    out_shape=jax.ShapeDtypeStruct((B, OUT), jnp.float32),
    scratch_shapes=[pltpu.VMEM((B, H), jnp.float32)],
    compiler_params=pltpu.CompilerParams(vmem_limit_bytes=128 * 1024 * 1024),
)


def kernel(x, sql, sql_table, input_table, Wg1, bg1, Wg2, bg2, Wf1, bf1,
           g1, be1, Wf2, bf2, g2, be2, Wp1, bp1, gp1, bep1, Wp2, bp2):
    xf = x.reshape(_NIDX).astype(jnp.int32)
    sf = sql.reshape(_NIDX).astype(jnp.int32)
    xrows = _build_sc_gather(DATA_NEMB)(xf, input_table)
    srows = _build_sc_gather(SQL_NEMB)(sf, sql_table)
    x_emb = xrows.reshape(B, IN_SZ)
    sql_emb = srows.reshape(B, G_IN)

    gates, loss = _gate_call(sql_emb, Wg1, bg1.reshape(1, H),
                             Wg2, bg2.reshape(1, K))
    out2 = _expert_call(gates, x_emb, Wf1, bf1.reshape(K, 1, H),
                        g1.reshape(K, 1, H), be1.reshape(K, 1, H), Wf2,
                        bf2.reshape(K, 1, H), g2.reshape(K, 1, H),
                        be2.reshape(K, 1, H), Wp1, bp1.reshape(1, H),
                        gp1.reshape(1, H), bep1.reshape(1, H), Wp2,
                        bp2.reshape(1, OUT))
    return out2.reshape(B), loss.reshape(())
